# baseline (speedup 1.0000x reference)
import functools

import jax
import jax.numpy as jnp
from jax import lax
from jax.experimental import pallas as pl
from jax.experimental.pallas import tpu as pltpu
from jax.experimental.pallas import tpu_sc as plsc

VOCAB = 100000
DIM = 128
SEQ = 200
BATCH = 4096
NCLS = 3
LANES = 16
NGRP = DIM // LANES
NC, NS = 2, 16
NW = NC * NS
RW = BATCH // NW
SPLIT = 96


NBUF = 4
GROUP = 64
UNROLL = 4


def _gather_sum_body(xr_hbm, tab_hbm, out_hbm,
                     idx_v, bufsA, bufsB, sums_v, sems):
    wid = lax.axis_index("s") * NC + lax.axis_index("c")

    def fire(r, k):
        pltpu.make_async_copy(
            tab_hbm.at[idx_v.at[r, pl.ds(0, SPLIT)]],
            bufsA[k].at[...], sems[k]).start()
        pltpu.make_async_copy(
            tab_hbm.at[idx_v.at[r, pl.ds(SPLIT, SEQ - SPLIT)]],
            bufsB[k].at[...], sems[k]).start()

    def drain(r, k):
        pltpu.make_async_copy(
            tab_hbm.at[idx_v.at[r, pl.ds(0, SPLIT)]],
            bufsA[k].at[...], sems[k]).wait()
        pltpu.make_async_copy(
            tab_hbm.at[idx_v.at[r, pl.ds(SPLIT, SEQ - SPLIT)]],
            bufsB[k].at[...], sems[k]).wait()

    def accumulate(r, k):
        bufA, bufB = bufsA[k], bufsB[k]

        def stepA(l, accs):
            for u in range(4):
                accs = tuple(
                    accs[g] + bufA[4 * l + u, g * LANES:(g + 1) * LANES]
                    for g in range(NGRP))
            return accs

        def stepB(l, accs):
            for u in range(4):
                accs = tuple(
                    accs[g] + bufB[4 * l + u, g * LANES:(g + 1) * LANES]
                    for g in range(NGRP))
            return accs

        z = jnp.zeros((LANES,), jnp.float32)
        accs = lax.fori_loop(0, SPLIT // 4, stepA, (z,) * NGRP)
        accs = lax.fori_loop(0, (SEQ - SPLIT) // 4, stepB, accs)
        for g in range(NGRP):
            sums_v[r, g * LANES:(g + 1) * LANES] = accs[g]

    def group_body(grp, _):
        pltpu.sync_copy(xr_hbm.at[wid, pl.ds(grp * GROUP, GROUP)], idx_v)
        for p in range(NBUF - 1):
            fire(p, p)

        def inner(i, _):
            for u in range(UNROLL):
                r = i * UNROLL + u

                @pl.when(r + NBUF - 1 < GROUP)
                def _():
                    fire(r + NBUF - 1, (u + NBUF - 1) % NBUF)

                drain(r, u)
                accumulate(r, u)
            return 0

        lax.fori_loop(0, GROUP // UNROLL, inner, 0)
        pltpu.sync_copy(sums_v,
                        out_hbm.at[pl.ds(wid * RW + grp * GROUP, GROUP)])
        return 0

    lax.fori_loop(0, RW // GROUP, group_body, 0)


def _gather_sum(xr, table):
    mesh = plsc.VectorSubcoreMesh(core_axis_name="c", subcore_axis_name="s",
                                  num_cores=NC)
    f = functools.partial(
        pl.kernel,
        out_type=jax.ShapeDtypeStruct((BATCH, DIM), jnp.float32),
        mesh=mesh,
        scratch_types=[
            pltpu.VMEM((GROUP, SEQ), jnp.int32),
            [pltpu.VMEM((SPLIT, DIM), jnp.float32) for _ in range(NBUF)],
            [pltpu.VMEM((SEQ - SPLIT, DIM), jnp.float32)
             for _ in range(NBUF)],
            pltpu.VMEM((GROUP, DIM), jnp.float32),
            [pltpu.SemaphoreType.DMA for _ in range(NBUF)],
        ],
        compiler_params=pltpu.CompilerParams(
            use_tc_tiling_on_sc=False,
            disable_bounds_checks=True,
            disable_semaphore_checks=True,
        ),
    )(_gather_sum_body)
    return f(xr, table)


def _finalize_body(s_ref, w_ref, b_ref, o_ref):
    pooled = s_ref[...] / jnp.float32(SEQ)
    l = jnp.dot(pooled, w_ref[...],
                preferred_element_type=jnp.float32) + b_ref[...]
    m = jnp.max(l, axis=0, keepdims=True)
    e = jnp.exp(l - m)
    tot = jnp.sum(e, axis=0, keepdims=True)
    y = e / tot
    lane = lax.broadcasted_iota(jnp.int32, (BATCH, NCLS), 1)
    best = jnp.max(y, axis=1, keepdims=True)
    pick = jnp.where(y == best, lane, jnp.int32(NCLS))
    o_ref[...] = jnp.min(pick, axis=1, keepdims=True)


def _finalize(sums, W, b):
    return pl.pallas_call(
        _finalize_body,
        in_specs=[
            pl.BlockSpec((BATCH, DIM), lambda: (0, 0)),
            pl.BlockSpec((DIM, NCLS), lambda: (0, 0)),
            pl.BlockSpec((1, NCLS), lambda: (0, 0)),
        ],
        out_specs=pl.BlockSpec((BATCH, 1), lambda: (0, 0)),
        out_shape=jax.ShapeDtypeStruct((BATCH, 1), jnp.int32),
    )(sums, W, b)


def kernel(x, table, W, b):
    xr = x.astype(jnp.int32).reshape(NW, RW, SEQ)
    sums = _gather_sum(xr, table)
    out = _finalize(sums, W, b.reshape(1, NCLS))
    return out.reshape(BATCH)


# speedup vs baseline: 1.0129x; 1.0129x over previous
import functools

import jax
import jax.numpy as jnp
from jax import lax
from jax.experimental import pallas as pl
from jax.experimental.pallas import tpu as pltpu
from jax.experimental.pallas import tpu_sc as plsc

VOCAB = 100000
DIM = 128
SEQ = 200
BATCH = 4096
NCLS = 3
LANES = 16
NGRP = DIM // LANES
NC, NS = 2, 16
NW = NC * NS
RW = BATCH // NW
SPLIT = 96


NBUF = 4
GROUP = 64
UNROLL = 4


def _gather_sum_body(xr_hbm, tab_hbm, out_hbm,
                     idx_v, bufsA, bufsB, sums_v, sems):
    wid = lax.axis_index("s") * NC + lax.axis_index("c")

    def fire(r, k):
        pltpu.make_async_copy(
            tab_hbm.at[idx_v.at[r, pl.ds(0, SPLIT)]],
            bufsA[k].at[...], sems[k]).start()
        pltpu.make_async_copy(
            tab_hbm.at[idx_v.at[r, pl.ds(SPLIT, SEQ - SPLIT)]],
            bufsB[k].at[...], sems[k]).start()

    def drain(r, k):
        pltpu.make_async_copy(
            tab_hbm.at[idx_v.at[r, pl.ds(0, SPLIT)]],
            bufsA[k].at[...], sems[k]).wait()
        pltpu.make_async_copy(
            tab_hbm.at[idx_v.at[r, pl.ds(SPLIT, SEQ - SPLIT)]],
            bufsB[k].at[...], sems[k]).wait()

    def accumulate(r, k):
        bufA, bufB = bufsA[k], bufsB[k]

        def stepA(l, accs):
            for u in range(4):
                accs = tuple(
                    accs[g] + bufA[4 * l + u, g * LANES:(g + 1) * LANES]
                    for g in range(NGRP))
            return accs

        def stepB(l, accs):
            for u in range(4):
                accs = tuple(
                    accs[g] + bufB[4 * l + u, g * LANES:(g + 1) * LANES]
                    for g in range(NGRP))
            return accs

        z = jnp.zeros((LANES,), jnp.float32)
        accs = lax.fori_loop(0, SPLIT // 4, stepA, (z,) * NGRP)
        accs = lax.fori_loop(0, (SEQ - SPLIT) // 4, stepB, accs)
        for g in range(NGRP):
            sums_v[r, g * LANES:(g + 1) * LANES] = accs[g]

    def group_body(grp, _):
        pltpu.sync_copy(
            xr_hbm.at[pl.ds(wid * RW + grp * GROUP, GROUP)], idx_v)
        for p in range(NBUF - 1):
            fire(p, p)

        def inner(i, _):
            for u in range(UNROLL):
                r = i * UNROLL + u

                @pl.when(r + NBUF - 1 < GROUP)
                def _():
                    fire(r + NBUF - 1, (u + NBUF - 1) % NBUF)

                drain(r, u)
                accumulate(r, u)
            return 0

        lax.fori_loop(0, GROUP // UNROLL, inner, 0)
        pltpu.sync_copy(sums_v,
                        out_hbm.at[pl.ds(wid * RW + grp * GROUP, GROUP)])
        return 0

    lax.fori_loop(0, RW // GROUP, group_body, 0)


def _gather_sum(x2d, table):
    mesh = plsc.VectorSubcoreMesh(core_axis_name="c", subcore_axis_name="s",
                                  num_cores=NC)
    f = functools.partial(
        pl.kernel,
        out_type=jax.ShapeDtypeStruct((BATCH, DIM), jnp.float32),
        mesh=mesh,
        scratch_types=[
            pltpu.VMEM((GROUP, SEQ), jnp.int32),
            [pltpu.VMEM((SPLIT, DIM), jnp.float32) for _ in range(NBUF)],
            [pltpu.VMEM((SEQ - SPLIT, DIM), jnp.float32)
             for _ in range(NBUF)],
            pltpu.VMEM((GROUP, DIM), jnp.float32),
            [pltpu.SemaphoreType.DMA for _ in range(NBUF)],
        ],
        compiler_params=pltpu.CompilerParams(use_tc_tiling_on_sc=False),
    )(_gather_sum_body)
    return f(x2d, table)


def _finalize_body(s_ref, w_ref, b_ref, o_ref):
    pooled = s_ref[...] / jnp.float32(SEQ)
    l = jnp.dot(pooled, w_ref[...],
                preferred_element_type=jnp.float32) + b_ref[...]
    m = jnp.max(l, axis=0, keepdims=True)
    e = jnp.exp(l - m)
    tot = jnp.sum(e, axis=0, keepdims=True)
    y = e / tot
    lane = lax.broadcasted_iota(jnp.int32, (BATCH, NCLS), 1)
    best = jnp.max(y, axis=1, keepdims=True)
    pick = jnp.where(y == best, lane, jnp.int32(NCLS))
    o_ref[...] = jnp.min(pick, axis=1)


def _finalize(sums, W, b):
    return pl.pallas_call(
        _finalize_body,
        in_specs=[
            pl.BlockSpec((BATCH, DIM), lambda: (0, 0)),
            pl.BlockSpec((DIM, NCLS), lambda: (0, 0)),
            pl.BlockSpec((1, NCLS), lambda: (0, 0)),
        ],
        out_specs=pl.BlockSpec((BATCH,), lambda: (0,)),
        out_shape=jax.ShapeDtypeStruct((BATCH,), jnp.int32),
    )(sums, W, b)


def kernel(x, table, W, b):
    sums = _gather_sum(x.astype(jnp.int32), table)
    return _finalize(sums, W, b.reshape(1, NCLS))


# speedup vs baseline: 1.0346x; 1.0215x over previous
import functools

import jax
import jax.numpy as jnp
from jax import lax
from jax.experimental import pallas as pl
from jax.experimental.pallas import tpu as pltpu
from jax.experimental.pallas import tpu_sc as plsc

VOCAB = 100000
DIM = 128
SEQ = 200
BATCH = 4096
NCLS = 3
LANES = 16
NGRP = DIM // LANES
NC, NS = 2, 16
NW = NC * NS
RW = BATCH // NW
SPLIT = 96


NBUF = 4
GROUP = 32
NGROUPS = RW // GROUP
UNROLL = 4


def _gather_sum_body(xr_hbm, tab_hbm, out_hbm,
                     idxs, bufsA, bufsB, sums_v, sems, sem_idx):
    wid = lax.axis_index("s") * NC + lax.axis_index("c")

    def fire(idx_v, r, k):
        pltpu.make_async_copy(
            tab_hbm.at[idx_v.at[r, pl.ds(0, SPLIT)]],
            bufsA[k].at[...], sems[k]).start()
        pltpu.make_async_copy(
            tab_hbm.at[idx_v.at[r, pl.ds(SPLIT, SEQ - SPLIT)]],
            bufsB[k].at[...], sems[k]).start()

    def drain(idx_v, r, k):
        pltpu.make_async_copy(
            tab_hbm.at[idx_v.at[r, pl.ds(0, SPLIT)]],
            bufsA[k].at[...], sems[k]).wait()
        pltpu.make_async_copy(
            tab_hbm.at[idx_v.at[r, pl.ds(SPLIT, SEQ - SPLIT)]],
            bufsB[k].at[...], sems[k]).wait()

    def accumulate(r, k):
        bufA, bufB = bufsA[k], bufsB[k]

        def stepA(l, accs):
            for u in range(4):
                accs = tuple(
                    accs[g] + bufA[4 * l + u, g * LANES:(g + 1) * LANES]
                    for g in range(NGRP))
            return accs

        def stepB(l, accs):
            for u in range(4):
                accs = tuple(
                    accs[g] + bufB[4 * l + u, g * LANES:(g + 1) * LANES]
                    for g in range(NGRP))
            return accs

        z = jnp.zeros((LANES,), jnp.float32)
        accs = lax.fori_loop(0, SPLIT // 4, stepA, (z,) * NGRP)
        accs = lax.fori_loop(0, (SEQ - SPLIT) // 4, stepB, accs)
        for g in range(NGRP):
            sums_v[r, g * LANES:(g + 1) * LANES] = accs[g]

    def idx_src(grp):
        return xr_hbm.at[pl.ds(wid * RW + grp * GROUP, GROUP)]

    def half(j, gbase, idx_cur, idx_nxt, has_next, is_first):
        if has_next is not None:
            @pl.when(has_next)
            def _():
                pltpu.make_async_copy(idx_src(gbase + 1), idx_nxt,
                                      sem_idx).start()

        if is_first is not None:
            @pl.when(is_first)
            def _():
                for p in range(NBUF - 1):
                    fire(idx_cur, p, p)

        def inner(i, _):
            for u in range(UNROLL):
                r = i * UNROLL + u
                fire(idx_cur, r + NBUF - 1, (u + NBUF - 1) % NBUF)
                drain(idx_cur, r, u)
                accumulate(r, u)
            return 0

        lax.fori_loop(0, GROUP // UNROLL - 1, inner, 0)

        for u in range(UNROLL):
            r = GROUP - UNROLL + u
            if u == 0:
                fire(idx_cur, GROUP - 1, (GROUP - 1) % NBUF)
            else:
                if has_next is not None:
                    if u == 1:
                        @pl.when(has_next)
                        def _():
                            pltpu.make_async_copy(idx_src(gbase + 1),
                                                  idx_nxt, sem_idx).wait()

                    @pl.when(has_next)
                    def _():
                        fire(idx_nxt, u - 1, (u + NBUF - 1) % NBUF)
            drain(idx_cur, r, u)
            accumulate(r, u)

        pltpu.sync_copy(sums_v,
                        out_hbm.at[pl.ds(wid * RW + gbase * GROUP, GROUP)])

    pltpu.sync_copy(idx_src(0), idxs[0])

    def pair_body(j, _):
        ga = 2 * j
        half(j, ga, idxs[0], idxs[1], ga + 1 < NGROUPS, j == 0)
        half(j, ga + 1, idxs[1], idxs[0], ga + 2 < NGROUPS, None)
        return 0

    lax.fori_loop(0, NGROUPS // 2, pair_body, 0)


def _gather_sum(x2d, table):
    mesh = plsc.VectorSubcoreMesh(core_axis_name="c", subcore_axis_name="s",
                                  num_cores=NC)
    f = functools.partial(
        pl.kernel,
        out_type=jax.ShapeDtypeStruct((BATCH, DIM), jnp.float32),
        mesh=mesh,
        scratch_types=[
            [pltpu.VMEM((GROUP, SEQ), jnp.int32) for _ in range(2)],
            [pltpu.VMEM((SPLIT, DIM), jnp.float32) for _ in range(NBUF)],
            [pltpu.VMEM((SEQ - SPLIT, DIM), jnp.float32)
             for _ in range(NBUF)],
            pltpu.VMEM((GROUP, DIM), jnp.float32),
            [pltpu.SemaphoreType.DMA for _ in range(NBUF)],
            pltpu.SemaphoreType.DMA,
        ],
        compiler_params=pltpu.CompilerParams(use_tc_tiling_on_sc=False),
    )(_gather_sum_body)
    return f(x2d, table)


def _finalize_body(s_ref, w_ref, b_ref, o_ref):
    pooled = s_ref[...] / jnp.float32(SEQ)
    l = jnp.dot(pooled, w_ref[...],
                preferred_element_type=jnp.float32) + b_ref[...]
    m = jnp.max(l, axis=0, keepdims=True)
    e = jnp.exp(l - m)
    tot = jnp.sum(e, axis=0, keepdims=True)
    y = e / tot
    lane = lax.broadcasted_iota(jnp.int32, (BATCH, NCLS), 1)
    best = jnp.max(y, axis=1, keepdims=True)
    pick = jnp.where(y == best, lane, jnp.int32(NCLS))
    o_ref[...] = jnp.min(pick, axis=1)


def _finalize(sums, W, b):
    return pl.pallas_call(
        _finalize_body,
        in_specs=[
            pl.BlockSpec((BATCH, DIM), lambda: (0, 0)),
            pl.BlockSpec((DIM, NCLS), lambda: (0, 0)),
            pl.BlockSpec((1, NCLS), lambda: (0, 0)),
        ],
        out_specs=pl.BlockSpec((BATCH,), lambda: (0,)),
        out_shape=jax.ShapeDtypeStruct((BATCH,), jnp.int32),
    )(sums, W, b)


def kernel(x, table, W, b):
    sums = _gather_sum(x.astype(jnp.int32), table)
    return _finalize(sums, W, b.reshape(1, NCLS))
